# baseline (speedup 1.0000x reference)
import functools

import jax
import jax.numpy as jnp
from jax import lax
from jax.experimental import pallas as pl
from jax.experimental.pallas import tpu as pltpu
from jax.experimental.pallas import tpu_sc as plsc

_NC, _NS = 2, 16
_NW = _NC * _NS
_CH = 64
_TQ = 256


def _routing_body(xq_ref, ek_ref, tc_ref, idx_ref):
    xq = xq_ref[...]
    ek = ek_ref[...]
    tc = tc_ref[...]
    nk = ek / jnp.clip(
        jnp.sqrt(jnp.sum(ek * ek, axis=1, keepdims=True)), 1e-12, None)
    qn = xq / jnp.clip(
        jnp.sqrt(jnp.sum(xq * xq, axis=1, keepdims=True)), 1e-12, None)
    cos = lax.dot_general(
        qn, nk, (((1,), (1,)), ((), ())),
        preferred_element_type=jnp.float32)
    scaled = (1.0 - cos) * (tc + 1.0)
    B, E = scaled.shape
    col = lax.broadcasted_iota(jnp.int32, (B, E), 1)
    m0 = jnp.min(scaled, axis=1, keepdims=True)
    i0 = jnp.min(jnp.where(scaled == m0, col, E), axis=1, keepdims=True)
    masked = jnp.where(col == i0, jnp.inf, scaled)
    m1 = jnp.min(masked, axis=1, keepdims=True)
    i1 = jnp.min(jnp.where(masked == m1, col, E), axis=1, keepdims=True)
    c = lax.broadcasted_iota(jnp.int32, (B, 24), 1)
    half = c // 12
    j = c % 12
    erow = jnp.where(j < 4, i0 * 8 + half * 4 + j,
                     i1 * 8 + half * 4 + (j - 4))
    grow = E * 8 + half * 4 + (j - 8)
    idx_ref[...] = jnp.where(j < 8, erow, grow)


def _routing_call(xq, ek, tc2d):
    B = xq.shape[0]
    return pl.pallas_call(
        _routing_body,
        out_shape=jax.ShapeDtypeStruct((B, 24), jnp.int32),
    )(xq, ek, tc2d)


@functools.lru_cache(maxsize=None)
def _gather_kernel(B, D):
    tiles_per_half = 12 * (B // _TQ)
    tpw = tiles_per_half // (_NW // 2)
    ipw = tpw * _TQ
    nchunks = tpw * (_TQ // _CH)
    mesh = plsc.VectorSubcoreMesh(core_axis_name="c", subcore_axis_name="s")

    @functools.partial(
        pl.kernel,
        out_type=(jax.ShapeDtypeStruct((12, B, D), jnp.float32),
                  jax.ShapeDtypeStruct((12, B, D), jnp.float32)),
        mesh=mesh,
        scratch_types=[
            pltpu.VMEM((ipw,), jnp.int32),
            pltpu.VMEM((2, _CH, D), jnp.float32),
            [pltpu.SemaphoreType.DMA] * 2,
            [pltpu.SemaphoreType.DMA] * 2,
        ],
    )
    def k(tab_hbm, idxk_hbm, idxv_hbm, pk_hbm, pv_hbm,
          idx_v, ebuf, gsems, wsems):
        wid = lax.axis_index("s") * _NC + lax.axis_index("c")

        def run(out_hbm, idxf_hbm, wloc):
            fbase = pl.multiple_of(wloc * ipw, ipw)
            pltpu.sync_copy(idxf_hbm.at[pl.ds(fbase, ipw)], idx_v)

            def dst(kc):
                tile = tpw * wloc + kc // (_TQ // _CH)
                c = tile // (B // _TQ)
                q0 = (tile % (B // _TQ)) * _TQ + (kc % (_TQ // _CH)) * _CH
                return c, pl.multiple_of(q0, _CH)

            def issue_gather(kc, s):
                off = pl.multiple_of(kc * _CH, _CH)
                pltpu.async_copy(tab_hbm.at[idx_v.at[pl.ds(off, _CH)]],
                                 ebuf.at[s], gsems[s])

            def wait_gather(s):
                pltpu.make_async_copy(tab_hbm.at[idx_v.at[pl.ds(0, _CH)]],
                                      ebuf.at[s], gsems[s]).wait()

            def issue_write(kc, s):
                c, q0 = dst(kc)
                pltpu.async_copy(ebuf.at[s], out_hbm.at[c, pl.ds(q0, _CH)],
                                 wsems[s])

            def wait_write(s):
                pltpu.make_async_copy(ebuf.at[s],
                                      out_hbm.at[0, pl.ds(0, _CH)],
                                      wsems[s]).wait()

            issue_gather(0, 0)
            issue_gather(1, 1)

            @pl.loop(0, nchunks, step=2)
            def _(k0):
                for s in range(2):
                    wait_gather(s)
                    issue_write(k0 + s, s)
                for s in range(2):
                    wait_write(s)
                    nxt = k0 + 2 + s

                    @pl.when(nxt < nchunks)
                    def _issue(nxt=nxt, s=s):
                        issue_gather(nxt, s)

        @pl.when(wid < _NW // 2)
        def _():
            run(pk_hbm, idxk_hbm, wid)

        @pl.when(wid >= _NW // 2)
        def _():
            run(pv_hbm, idxv_hbm, wid - _NW // 2)

    return k


def kernel(x_querry, l, x_block, e_k, e_p_0, g_p_0, train_count):
    B, _ = x_querry.shape
    E, PLEN, D = e_p_0.shape
    idx = _routing_call(x_querry, e_k, train_count.reshape(1, E))
    idx_t = idx.T
    idxk = idx_t[:12].reshape(-1)
    idxv = idx_t[12:].reshape(-1)
    table = jnp.concatenate([e_p_0.reshape(E * PLEN, D), g_p_0], axis=0)
    pk, pv = _gather_kernel(B, D)(table, idxk, idxv)
    return pk.transpose(1, 0, 2), pv.transpose(1, 0, 2), x_block


# speedup vs baseline: 2.0962x; 2.0962x over previous
import functools

import jax
import jax.numpy as jnp
from jax import lax
from jax.experimental import pallas as pl
from jax.experimental.pallas import tpu as pltpu
from jax.experimental.pallas import tpu_sc as plsc

_NC, _NS = 2, 16
_NW = _NC * _NS
_CH = 64
_TQ = 256


def _routing_body(xq_ref, ek_ref, tc_ref, idx_ref):
    xq = xq_ref[...]
    ek = ek_ref[...]
    tc = tc_ref[...]
    nk = ek / jnp.clip(
        jnp.sqrt(jnp.sum(ek * ek, axis=1, keepdims=True)), 1e-12, None)
    qn = xq / jnp.clip(
        jnp.sqrt(jnp.sum(xq * xq, axis=1, keepdims=True)), 1e-12, None)
    cos = lax.dot_general(
        qn, nk, (((1,), (1,)), ((), ())),
        preferred_element_type=jnp.float32)
    scaled = (1.0 - cos) * (tc + 1.0)
    B, E = scaled.shape
    col = lax.broadcasted_iota(jnp.int32, (B, E), 1)
    m0 = jnp.min(scaled, axis=1, keepdims=True)
    i0 = jnp.min(jnp.where(scaled == m0, col, E), axis=1, keepdims=True)
    masked = jnp.where(col == i0, jnp.inf, scaled)
    m1 = jnp.min(masked, axis=1, keepdims=True)
    i1 = jnp.min(jnp.where(masked == m1, col, E), axis=1, keepdims=True)
    c = lax.broadcasted_iota(jnp.int32, (B, 24), 1)
    b = lax.broadcasted_iota(jnp.int32, (B, 24), 0)
    half = c // 12
    j = c % 12
    erow = jnp.where(j < 4, i0 * 8 + half * 4 + j,
                     i1 * 8 + half * 4 + (j - 4))
    grow = E * 8 + (b % 64) * 8 + half * 4 + (j - 8)
    idx_ref[...] = jnp.where(j < 8, erow, grow)


def _routing_call(xq, ek, tc2d):
    B = xq.shape[0]
    return pl.pallas_call(
        _routing_body,
        out_shape=jax.ShapeDtypeStruct((B, 24), jnp.int32),
    )(xq, ek, tc2d)


@functools.lru_cache(maxsize=None)
def _gather_kernel(B, D):
    tiles_per_half = 12 * (B // _TQ)
    tpw = tiles_per_half // (_NW // 2)
    ipw = tpw * _TQ
    nchunks = tpw * (_TQ // _CH)
    mesh = plsc.VectorSubcoreMesh(core_axis_name="c", subcore_axis_name="s")

    @functools.partial(
        pl.kernel,
        out_type=(jax.ShapeDtypeStruct((12, B, D), jnp.float32),
                  jax.ShapeDtypeStruct((12, B, D), jnp.float32)),
        mesh=mesh,
        scratch_types=[
            pltpu.VMEM((ipw,), jnp.int32),
            pltpu.VMEM((2, _CH, D), jnp.float32),
            [pltpu.SemaphoreType.DMA] * 2,
            [pltpu.SemaphoreType.DMA] * 2,
        ],
    )
    def k(tab_hbm, idxk_hbm, idxv_hbm, pk_hbm, pv_hbm,
          idx_v, ebuf, gsems, wsems):
        wid = lax.axis_index("s") * _NC + lax.axis_index("c")

        def run(out_hbm, idxf_hbm, wloc):
            fbase = pl.multiple_of(wloc * ipw, ipw)
            pltpu.sync_copy(idxf_hbm.at[pl.ds(fbase, ipw)], idx_v)

            def dst(kc):
                tile = tpw * wloc + kc // (_TQ // _CH)
                c = tile // (B // _TQ)
                q0 = (tile % (B // _TQ)) * _TQ + (kc % (_TQ // _CH)) * _CH
                return c, pl.multiple_of(q0, _CH)

            def issue_gather(kc, s):
                off = pl.multiple_of(kc * _CH, _CH)
                pltpu.async_copy(tab_hbm.at[idx_v.at[pl.ds(off, _CH)]],
                                 ebuf.at[s], gsems[s])

            def wait_gather(s):
                pltpu.make_async_copy(tab_hbm.at[idx_v.at[pl.ds(0, _CH)]],
                                      ebuf.at[s], gsems[s]).wait()

            def issue_write(kc, s):
                c, q0 = dst(kc)
                pltpu.async_copy(ebuf.at[s], out_hbm.at[c, pl.ds(q0, _CH)],
                                 wsems[s])

            def wait_write(s):
                pltpu.make_async_copy(ebuf.at[s],
                                      out_hbm.at[0, pl.ds(0, _CH)],
                                      wsems[s]).wait()

            issue_gather(0, 0)
            issue_gather(1, 1)

            @pl.loop(0, nchunks, step=2)
            def _(k0):
                for s in range(2):
                    wait_gather(s)
                    issue_write(k0 + s, s)
                for s in range(2):
                    wait_write(s)
                    nxt = k0 + 2 + s

                    @pl.when(nxt < nchunks)
                    def _issue(nxt=nxt, s=s):
                        issue_gather(nxt, s)

        @pl.when(wid < _NW // 2)
        def _():
            run(pk_hbm, idxk_hbm, wid)

        @pl.when(wid >= _NW // 2)
        def _():
            run(pv_hbm, idxv_hbm, wid - _NW // 2)

    return k


def kernel(x_querry, l, x_block, e_k, e_p_0, g_p_0, train_count):
    B, _ = x_querry.shape
    E, PLEN, D = e_p_0.shape
    idx = _routing_call(x_querry, e_k, train_count.reshape(1, E))
    idx_t = idx.T
    idxk = idx_t[:12].reshape(-1)
    idxv = idx_t[12:].reshape(-1)
    table = jnp.concatenate(
        [e_p_0.reshape(E * PLEN, D), jnp.tile(g_p_0, (64, 1))], axis=0)
    pk, pv = _gather_kernel(B, D)(table, idxk, idxv)
    return pk.transpose(1, 0, 2), pv.transpose(1, 0, 2), x_block


# speedup vs baseline: 2.2552x; 1.0758x over previous
import functools

import jax
import jax.numpy as jnp
from jax import lax
from jax.experimental import pallas as pl
from jax.experimental.pallas import tpu as pltpu
from jax.experimental.pallas import tpu_sc as plsc

_NC, _NS = 2, 16
_NW = _NC * _NS
_CH = 64
_TQ = 256


def _routing_body(xq_ref, ek_ref, tc_ref, idx_ref):
    xq = xq_ref[...]
    ek = ek_ref[...]
    tc = tc_ref[...]
    nk = ek / jnp.clip(
        jnp.sqrt(jnp.sum(ek * ek, axis=1, keepdims=True)), 1e-12, None)
    qn = xq / jnp.clip(
        jnp.sqrt(jnp.sum(xq * xq, axis=1, keepdims=True)), 1e-12, None)
    cos = lax.dot_general(
        qn, nk, (((1,), (1,)), ((), ())),
        preferred_element_type=jnp.float32)
    scaled = (1.0 - cos) * (tc + 1.0)
    B, E = scaled.shape
    col = lax.broadcasted_iota(jnp.int32, (B, E), 1)
    m0 = jnp.min(scaled, axis=1, keepdims=True)
    i0 = jnp.min(jnp.where(scaled == m0, col, E), axis=1, keepdims=True)
    masked = jnp.where(col == i0, jnp.inf, scaled)
    m1 = jnp.min(masked, axis=1, keepdims=True)
    i1 = jnp.min(jnp.where(masked == m1, col, E), axis=1, keepdims=True)
    c = lax.broadcasted_iota(jnp.int32, (B, 24), 1)
    b = lax.broadcasted_iota(jnp.int32, (B, 24), 0)
    half = c // 12
    j = c % 12
    erow = jnp.where(j < 4, i0 * 8 + half * 4 + j,
                     i1 * 8 + half * 4 + (j - 4))
    grow = (b % 64) * 8 + half * 4 + (j - 8)
    idx_ref[...] = jnp.where(j < 8, erow, grow)


def _routing_call(xq, ek, tc2d):
    B = xq.shape[0]
    return pl.pallas_call(
        _routing_body,
        out_shape=jax.ShapeDtypeStruct((B, 24), jnp.int32),
    )(xq, ek, tc2d)


@functools.lru_cache(maxsize=None)
def _gather_kernel(B, D):
    nt = B // _TQ
    ne = 8 * nt // (_NW // 2)
    nchunks = ne * (_TQ // _CH)
    ipw = nchunks * _CH + _CH
    mesh = plsc.VectorSubcoreMesh(core_axis_name="c", subcore_axis_name="s")

    @functools.partial(
        pl.kernel,
        out_type=(jax.ShapeDtypeStruct((12, B, D), jnp.float32),
                  jax.ShapeDtypeStruct((12, B, D), jnp.float32)),
        mesh=mesh,
        scratch_types=[
            pltpu.VMEM((ipw,), jnp.int32),
            pltpu.VMEM((2, _CH, D), jnp.float32),
            [pltpu.SemaphoreType.DMA] * 2,
            [pltpu.SemaphoreType.DMA] * 2,
        ],
    )
    def k(ep_hbm, gr_hbm, idxk_hbm, idxv_hbm, pk_hbm, pv_hbm,
          idx_v, ebuf, gsems, wsems):
        wid = lax.axis_index("s") * _NC + lax.axis_index("c")

        def run(out_hbm, idxf_hbm, wloc):
            ebase = pl.multiple_of(wloc * (ne * _TQ), _CH)
            pltpu.sync_copy(idxf_hbm.at[pl.ds(ebase, ne * _TQ)],
                            idx_v.at[pl.ds(0, ne * _TQ)])
            gbase = pl.multiple_of(8 * nt * _TQ + wloc * _TQ, _CH)
            pltpu.sync_copy(idxf_hbm.at[pl.ds(gbase, _CH)],
                            idx_v.at[pl.ds(ne * _TQ, _CH)])

            def dst(kc):
                tile = ne * wloc + kc // (_TQ // _CH)
                c = tile // nt
                q0 = (tile % nt) * _TQ + (kc % (_TQ // _CH)) * _CH
                return c, pl.multiple_of(q0, _CH)

            def issue_gather(kc, s):
                off = pl.multiple_of(kc * _CH, _CH)
                pltpu.async_copy(ep_hbm.at[idx_v.at[pl.ds(off, _CH)]],
                                 ebuf.at[s], gsems[s])

            def wait_gather(s):
                pltpu.make_async_copy(ep_hbm.at[idx_v.at[pl.ds(0, _CH)]],
                                      ebuf.at[s], gsems[s]).wait()

            def issue_write(kc, s):
                c, q0 = dst(kc)
                pltpu.async_copy(ebuf.at[s], out_hbm.at[c, pl.ds(q0, _CH)],
                                 wsems[s])

            def wait_write(s):
                pltpu.make_async_copy(ebuf.at[s],
                                      out_hbm.at[0, pl.ds(0, _CH)],
                                      wsems[s]).wait()

            issue_gather(0, 0)
            issue_gather(1, 1)

            @pl.loop(0, nchunks, step=2)
            def _(k0):
                for s in range(2):
                    wait_gather(s)
                    issue_write(k0 + s, s)
                for s in range(2):
                    wait_write(s)
                    nxt = k0 + 2 + s

                    @pl.when(nxt < nchunks)
                    def _issue(nxt=nxt, s=s):
                        issue_gather(nxt, s)

            goff = pl.multiple_of(ne * _TQ, _CH)
            pltpu.async_copy(gr_hbm.at[idx_v.at[pl.ds(goff, _CH)]],
                             ebuf.at[0], gsems[0])
            wait_gather(0)
            cg = 8 + wloc // nt
            qg = pl.multiple_of((wloc % nt) * _TQ, _TQ)
            for i in range(_TQ // _CH):
                pltpu.async_copy(ebuf.at[0],
                                 out_hbm.at[cg, pl.ds(qg + i * _CH, _CH)],
                                 wsems[0])
            for i in range(_TQ // _CH):
                wait_write(0)

        @pl.when(wid < _NW // 2)
        def _():
            run(pk_hbm, idxk_hbm, wid)

        @pl.when(wid >= _NW // 2)
        def _():
            run(pv_hbm, idxv_hbm, wid - _NW // 2)

    return k


def kernel(x_querry, l, x_block, e_k, e_p_0, g_p_0, train_count):
    B, _ = x_querry.shape
    E, PLEN, D = e_p_0.shape
    idx = _routing_call(x_querry, e_k, train_count.reshape(1, E))
    idx_t = idx.T
    idxk = idx_t[:12].reshape(-1)
    idxv = idx_t[12:].reshape(-1)
    g_rep = jnp.tile(g_p_0, (64, 1))
    pk, pv = _gather_kernel(B, D)(
        e_p_0.reshape(E * PLEN, D), g_rep, idxk, idxv)
    return pk.transpose(1, 0, 2), pv.transpose(1, 0, 2), x_block


# speedup vs baseline: 2.2901x; 1.0155x over previous
import functools

import jax
import jax.numpy as jnp
from jax import lax
from jax.experimental import pallas as pl
from jax.experimental.pallas import tpu as pltpu
from jax.experimental.pallas import tpu_sc as plsc

_NC, _NS = 2, 16
_NW = _NC * _NS
_CH = 32
_NSLOT = 4
_TQ = 256


def _routing_body(xq_ref, ek_ref, tc_ref, idx_ref):
    xq = xq_ref[...]
    ek = ek_ref[...]
    tc = tc_ref[...]
    nk = ek / jnp.clip(
        jnp.sqrt(jnp.sum(ek * ek, axis=1, keepdims=True)), 1e-12, None)
    qn = xq / jnp.clip(
        jnp.sqrt(jnp.sum(xq * xq, axis=1, keepdims=True)), 1e-12, None)
    cos = lax.dot_general(
        qn, nk, (((1,), (1,)), ((), ())),
        preferred_element_type=jnp.float32)
    scaled = (1.0 - cos) * (tc + 1.0)
    B, E = scaled.shape
    col = lax.broadcasted_iota(jnp.int32, (B, E), 1)
    m0 = jnp.min(scaled, axis=1, keepdims=True)
    i0 = jnp.min(jnp.where(scaled == m0, col, E), axis=1, keepdims=True)
    masked = jnp.where(col == i0, jnp.inf, scaled)
    m1 = jnp.min(masked, axis=1, keepdims=True)
    i1 = jnp.min(jnp.where(masked == m1, col, E), axis=1, keepdims=True)
    c = lax.broadcasted_iota(jnp.int32, (B, 24), 1)
    b = lax.broadcasted_iota(jnp.int32, (B, 24), 0)
    half = c // 12
    j = c % 12
    erow = jnp.where(j < 4, i0 * 8 + half * 4 + j,
                     i1 * 8 + half * 4 + (j - 4))
    grow = (b % 32) * 8 + half * 4 + (j - 8)
    idx_ref[...] = jnp.where(j < 8, erow, grow)


def _routing_call(xq, ek, tc2d):
    B = xq.shape[0]
    return pl.pallas_call(
        _routing_body,
        out_shape=jax.ShapeDtypeStruct((B, 24), jnp.int32),
    )(xq, ek, tc2d)


@functools.lru_cache(maxsize=None)
def _gather_kernel(B, D):
    nt = B // _TQ
    ne = 8 * nt // (_NW // 2)
    nchunks = ne * (_TQ // _CH)
    ipw = nchunks * _CH + _CH
    mesh = plsc.VectorSubcoreMesh(core_axis_name="c", subcore_axis_name="s")

    @functools.partial(
        pl.kernel,
        out_type=(jax.ShapeDtypeStruct((12, B, D), jnp.float32),
                  jax.ShapeDtypeStruct((12, B, D), jnp.float32)),
        mesh=mesh,
        scratch_types=[
            pltpu.VMEM((ipw,), jnp.int32),
            pltpu.VMEM((_NSLOT, _CH, D), jnp.float32),
            [pltpu.SemaphoreType.DMA] * _NSLOT,
            [pltpu.SemaphoreType.DMA] * _NSLOT,
        ],
    )
    def k(ep_hbm, gr_hbm, idxk_hbm, idxv_hbm, pk_hbm, pv_hbm,
          idx_v, ebuf, gsems, wsems):
        wid = lax.axis_index("s") * _NC + lax.axis_index("c")

        def run(out_hbm, idxf_hbm, wloc):
            ebase = pl.multiple_of(wloc * (ne * _TQ), _CH)
            pltpu.sync_copy(idxf_hbm.at[pl.ds(ebase, ne * _TQ)],
                            idx_v.at[pl.ds(0, ne * _TQ)])
            gbase = pl.multiple_of(8 * nt * _TQ + wloc * _TQ, _CH)
            pltpu.sync_copy(idxf_hbm.at[pl.ds(gbase, _CH)],
                            idx_v.at[pl.ds(ne * _TQ, _CH)])

            def dst(kc):
                tile = ne * wloc + kc // (_TQ // _CH)
                c = tile // nt
                q0 = (tile % nt) * _TQ + (kc % (_TQ // _CH)) * _CH
                return c, pl.multiple_of(q0, _CH)

            def issue_gather(kc, s):
                off = pl.multiple_of(kc * _CH, _CH)
                pltpu.async_copy(ep_hbm.at[idx_v.at[pl.ds(off, _CH)]],
                                 ebuf.at[s], gsems[s])

            def wait_gather(s):
                pltpu.make_async_copy(ep_hbm.at[idx_v.at[pl.ds(0, _CH)]],
                                      ebuf.at[s], gsems[s]).wait()

            def issue_write(kc, s):
                c, q0 = dst(kc)
                pltpu.async_copy(ebuf.at[s], out_hbm.at[c, pl.ds(q0, _CH)],
                                 wsems[s])

            def wait_write(s):
                pltpu.make_async_copy(ebuf.at[s],
                                      out_hbm.at[0, pl.ds(0, _CH)],
                                      wsems[s]).wait()

            for s in range(_NSLOT):
                issue_gather(s, s)

            @pl.loop(0, nchunks, step=_NSLOT)
            def _(k0):
                for s in range(_NSLOT):
                    wait_gather(s)
                    issue_write(k0 + s, s)
                for s in range(_NSLOT):
                    wait_write(s)
                    nxt = k0 + _NSLOT + s

                    @pl.when(nxt < nchunks)
                    def _issue(nxt=nxt, s=s):
                        issue_gather(nxt, s)

            goff = pl.multiple_of(ne * _TQ, _CH)
            pltpu.async_copy(gr_hbm.at[idx_v.at[pl.ds(goff, _CH)]],
                             ebuf.at[0], gsems[0])
            wait_gather(0)
            cg = 8 + wloc // nt
            qg = pl.multiple_of((wloc % nt) * _TQ, _TQ)
            for i in range(_TQ // _CH):
                pltpu.async_copy(ebuf.at[0],
                                 out_hbm.at[cg, pl.ds(qg + i * _CH, _CH)],
                                 wsems[0])
            for i in range(_TQ // _CH):
                wait_write(0)

        @pl.when(wid < _NW // 2)
        def _():
            run(pk_hbm, idxk_hbm, wid)

        @pl.when(wid >= _NW // 2)
        def _():
            run(pv_hbm, idxv_hbm, wid - _NW // 2)

    return k


def kernel(x_querry, l, x_block, e_k, e_p_0, g_p_0, train_count):
    B, _ = x_querry.shape
    E, PLEN, D = e_p_0.shape
    idx = _routing_call(x_querry, e_k, train_count.reshape(1, E))
    idx_t = idx.T
    idxk = idx_t[:12].reshape(-1)
    idxv = idx_t[12:].reshape(-1)
    g_rep = jnp.tile(g_p_0, (_CH, 1))
    pk, pv = _gather_kernel(B, D)(
        e_p_0.reshape(E * PLEN, D), g_rep, idxk, idxv)
    return pk.transpose(1, 0, 2), pv.transpose(1, 0, 2), x_block
